# speedup vs baseline: 1.0066x; 1.0066x over previous
import functools

import jax
import jax.numpy as jnp
from jax import lax
from jax.experimental import pallas as pl
from jax.experimental.pallas import tpu as pltpu
from jax.experimental.pallas import tpu_sc as plsc

BATCH = 4
SEQ = 2048
EMB_DIM = 128
NUM_CORES = 2
NUM_SUBCORES = 16
NUM_WORKERS = NUM_CORES * NUM_SUBCORES
ROWS_PER_WORKER = BATCH * SEQ // NUM_WORKERS
IDX_BLOCKS = ROWS_PER_WORKER // 128
LANES = 16
COL_CHUNKS = EMB_DIM // LANES


def _emb_body(x_hbm, tok_hbm, pos_hbm, out_hbm, idx_v, rows_v, pos_v, sem):
    wid = lax.axis_index("s") * NUM_CORES + lax.axis_index("c")
    base = wid * ROWS_PER_WORKER

    pltpu.sync_copy(x_hbm.at[pl.ds(wid * IDX_BLOCKS, IDX_BLOCKS)], idx_v)

    copies = [
        pltpu.async_copy(
            tok_hbm.at[idx_v.at[j]], rows_v.at[pl.ds(j * 128, 128)], sem
        )
        for j in range(IDX_BLOCKS)
    ]
    pos_base = lax.rem(wid, SEQ // ROWS_PER_WORKER) * ROWS_PER_WORKER
    pltpu.sync_copy(pos_hbm.at[pl.ds(pos_base, ROWS_PER_WORKER)], pos_v)
    for cp in copies:
        cp.wait()

    def add_row(r, carry):
        for c in range(COL_CHUNKS):
            sl = pl.ds(c * LANES, LANES)
            rows_v[r, sl] = rows_v[r, sl] + pos_v[r, sl]
        return carry

    lax.fori_loop(0, ROWS_PER_WORKER, add_row, 0, unroll=2)

    pltpu.sync_copy(rows_v, out_hbm.at[pl.ds(base, ROWS_PER_WORKER)])


_emb_call = functools.partial(
    pl.kernel,
    out_type=jax.ShapeDtypeStruct((BATCH * SEQ, EMB_DIM), jnp.float32),
    mesh=plsc.VectorSubcoreMesh(core_axis_name="c", subcore_axis_name="s"),
    scratch_types=[
        pltpu.VMEM((IDX_BLOCKS, 128), jnp.int32),
        pltpu.VMEM((ROWS_PER_WORKER, EMB_DIM), jnp.float32),
        pltpu.VMEM((ROWS_PER_WORKER, EMB_DIM), jnp.float32),
        pltpu.SemaphoreType.DMA,
    ],
)(_emb_body)


def kernel(x, tok_emb, pos_emb):
    x2d = x.reshape(BATCH * SEQ // 128, 128).astype(jnp.int32)
    out = _emb_call(x2d, tok_emb, pos_emb)
    return out.reshape(BATCH, SEQ, EMB_DIM)


# speedup vs baseline: 1.3456x; 1.3367x over previous
import functools

import jax
import jax.numpy as jnp
from jax import lax
from jax.experimental import pallas as pl
from jax.experimental.pallas import tpu as pltpu
from jax.experimental.pallas import tpu_sc as plsc

BATCH = 4
SEQ = 2048
EMB_DIM = 128
NUM_CORES = 2
NUM_SUBCORES = 16
NUM_WORKERS = NUM_CORES * NUM_SUBCORES
ROWS_PER_WORKER = BATCH * SEQ // NUM_WORKERS
IDX_BLOCKS = ROWS_PER_WORKER // 128
LANES = 16
COL_CHUNKS = EMB_DIM // LANES


def _emb_body(x_hbm, tok_hbm, pos_hbm, out_hbm, idx_v, rows_v, sem):
    wid = lax.axis_index("s") * NUM_CORES + lax.axis_index("c")
    base = wid * ROWS_PER_WORKER

    pltpu.sync_copy(x_hbm.at[pl.ds(wid * IDX_BLOCKS, IDX_BLOCKS)], idx_v)

    pos_base = lax.rem(wid, SEQ // ROWS_PER_WORKER) * ROWS_PER_WORKER
    pltpu.sync_copy(pos_hbm.at[pl.ds(pos_base, ROWS_PER_WORKER)], rows_v)
    copies = [
        pltpu.async_copy(
            tok_hbm.at[idx_v.at[j]], rows_v.at[pl.ds(j * 128, 128)], sem,
            add=True,
        )
        for j in range(IDX_BLOCKS)
    ]
    for cp in copies:
        cp.wait()

    pltpu.sync_copy(rows_v, out_hbm.at[pl.ds(base, ROWS_PER_WORKER)])


_emb_call = functools.partial(
    pl.kernel,
    out_type=jax.ShapeDtypeStruct((BATCH * SEQ, EMB_DIM), jnp.float32),
    mesh=plsc.VectorSubcoreMesh(core_axis_name="c", subcore_axis_name="s"),
    scratch_types=[
        pltpu.VMEM((IDX_BLOCKS, 128), jnp.int32),
        pltpu.VMEM((ROWS_PER_WORKER, EMB_DIM), jnp.float32),
        pltpu.SemaphoreType.DMA,
    ],
)(_emb_body)


def kernel(x, tok_emb, pos_emb):
    x2d = x.reshape(BATCH * SEQ // 128, 128).astype(jnp.int32)
    out = _emb_call(x2d, tok_emb, pos_emb)
    return out.reshape(BATCH, SEQ, EMB_DIM)


# speedup vs baseline: 1.3533x; 1.0057x over previous
import functools

import jax
import jax.numpy as jnp
from jax import lax
from jax.experimental import pallas as pl
from jax.experimental.pallas import tpu as pltpu
from jax.experimental.pallas import tpu_sc as plsc

BATCH = 4
SEQ = 2048
EMB_DIM = 128
NUM_CORES = 2
NUM_SUBCORES = 16
NUM_WORKERS = NUM_CORES * NUM_SUBCORES
ROWS_PER_WORKER = BATCH * SEQ // NUM_WORKERS
NCHUNK = 4
CHUNK = ROWS_PER_WORKER // NCHUNK


def _emb_body(x_hbm, tok_hbm, pos_hbm, out_hbm, idx_v, rows_v, semp, semg, semo):
    wid = lax.axis_index("s") * NUM_CORES + lax.axis_index("c")
    base = wid * ROWS_PER_WORKER
    pos_base = lax.rem(wid, SEQ // ROWS_PER_WORKER) * ROWS_PER_WORKER

    pltpu.sync_copy(x_hbm.at[pl.ds(wid * NCHUNK, NCHUNK)], idx_v)

    pos_cps = [
        pltpu.async_copy(
            pos_hbm.at[pl.ds(pos_base + c * CHUNK, CHUNK)],
            rows_v.at[pl.ds(c * CHUNK, CHUNK)],
            semp.at[c],
        )
        for c in range(NCHUNK)
    ]
    g_cps = []
    for c in range(NCHUNK):
        pos_cps[c].wait()
        g_cps.append(
            pltpu.async_copy(
                tok_hbm.at[idx_v.at[c]],
                rows_v.at[pl.ds(c * CHUNK, CHUNK)],
                semg.at[c],
                add=True,
            )
        )
    o_cps = []
    for c in range(NCHUNK):
        g_cps[c].wait()
        o_cps.append(
            pltpu.async_copy(
                rows_v.at[pl.ds(c * CHUNK, CHUNK)],
                out_hbm.at[pl.ds(base + c * CHUNK, CHUNK)],
                semo.at[c],
            )
        )
    for cp in o_cps:
        cp.wait()


_emb_call = functools.partial(
    pl.kernel,
    out_type=jax.ShapeDtypeStruct((BATCH * SEQ, EMB_DIM), jnp.float32),
    mesh=plsc.VectorSubcoreMesh(core_axis_name="c", subcore_axis_name="s"),
    scratch_types=[
        pltpu.VMEM((NCHUNK, CHUNK), jnp.int32),
        pltpu.VMEM((ROWS_PER_WORKER, EMB_DIM), jnp.float32),
        pltpu.SemaphoreType.DMA((NCHUNK,)),
        pltpu.SemaphoreType.DMA((NCHUNK,)),
        pltpu.SemaphoreType.DMA((NCHUNK,)),
    ],
)(_emb_body)


def kernel(x, tok_emb, pos_emb):
    x2d = x.reshape(NUM_WORKERS * NCHUNK, CHUNK).astype(jnp.int32)
    out = _emb_call(x2d, tok_emb, pos_emb)
    return out.reshape(BATCH, SEQ, EMB_DIM)


# speedup vs baseline: 1.3541x; 1.0006x over previous
import functools

import jax
import jax.numpy as jnp
from jax import lax
from jax.experimental import pallas as pl
from jax.experimental.pallas import tpu as pltpu
from jax.experimental.pallas import tpu_sc as plsc

BATCH = 4
SEQ = 2048
EMB_DIM = 128
NUM_CORES = 2
NUM_SUBCORES = 16
NUM_WORKERS = NUM_CORES * NUM_SUBCORES
ROWS_PER_WORKER = BATCH * SEQ // NUM_WORKERS
NCHUNK = 4
CHUNK = ROWS_PER_WORKER // NCHUNK
POS_GROUPS = 4


def _emb_body(x_hbm, tok_hbm, pos_hbm, out_hbm, idx_v, rows_v, spos, semp, semg, semo):
    core = lax.axis_index("c")
    sub = lax.axis_index("s")
    wid = sub * NUM_CORES + core
    base = wid * ROWS_PER_WORKER
    group = lax.rem(sub, POS_GROUPS)

    @pl.when(sub < POS_GROUPS)
    def _stage():
        pos_base = (sub * NUM_CORES + core) * ROWS_PER_WORKER
        pltpu.sync_copy(
            pos_hbm.at[pl.ds(pos_base, ROWS_PER_WORKER)],
            spos.at[pl.ds(sub * ROWS_PER_WORKER, ROWS_PER_WORKER)],
        )

    pltpu.sync_copy(x_hbm.at[pl.ds(wid * NCHUNK, NCHUNK)], idx_v)
    plsc.subcore_barrier()

    pos_cps = [
        pltpu.async_copy(
            spos.at[pl.ds(group * ROWS_PER_WORKER + c * CHUNK, CHUNK)],
            rows_v.at[pl.ds(c * CHUNK, CHUNK)],
            semp.at[c],
        )
        for c in range(NCHUNK)
    ]
    g_cps = []
    for c in range(NCHUNK):
        pos_cps[c].wait()
        g_cps.append(
            pltpu.async_copy(
                tok_hbm.at[idx_v.at[c]],
                rows_v.at[pl.ds(c * CHUNK, CHUNK)],
                semg.at[c],
                add=True,
            )
        )
    o_cps = []
    for c in range(NCHUNK):
        g_cps[c].wait()
        o_cps.append(
            pltpu.async_copy(
                rows_v.at[pl.ds(c * CHUNK, CHUNK)],
                out_hbm.at[pl.ds(base + c * CHUNK, CHUNK)],
                semo.at[c],
            )
        )
    for cp in o_cps:
        cp.wait()


_emb_call = functools.partial(
    pl.kernel,
    out_type=jax.ShapeDtypeStruct((BATCH * SEQ, EMB_DIM), jnp.float32),
    mesh=plsc.VectorSubcoreMesh(core_axis_name="c", subcore_axis_name="s"),
    scratch_types=[
        pltpu.VMEM((NCHUNK, CHUNK), jnp.int32),
        pltpu.VMEM((ROWS_PER_WORKER, EMB_DIM), jnp.float32),
        pltpu.VMEM_SHARED((POS_GROUPS * ROWS_PER_WORKER, EMB_DIM), jnp.float32),
        pltpu.SemaphoreType.DMA((NCHUNK,)),
        pltpu.SemaphoreType.DMA((NCHUNK,)),
        pltpu.SemaphoreType.DMA((NCHUNK,)),
    ],
)(_emb_body)


def kernel(x, tok_emb, pos_emb):
    x2d = x.reshape(NUM_WORKERS * NCHUNK, CHUNK).astype(jnp.int32)
    out = _emb_call(x2d, tok_emb, pos_emb)
    return out.reshape(BATCH, SEQ, EMB_DIM)
